# Initial kernel scaffold; baseline (speedup 1.0000x reference)
import jax
import jax.numpy as jnp
from jax.experimental import pallas as pl


def kernel(hidden_states, gate_weight, w13_weight, w2_weight):
    raise NotImplementedError("write your pallas kernel here")


# speedup vs baseline: 3.2735x; 3.2735x over previous
import functools

import jax
import jax.numpy as jnp
from jax import lax
from jax.experimental import pallas as pl
from jax.experimental.pallas import tpu as pltpu
from jax.experimental.pallas import tpu_sc as plsc

S = 2048
H = 1024
E = 8
I = 2048

BLK = 256
NBLK = (2 * S + E * (BLK - 1) + BLK - 1) // BLK
CAP = NBLK * BLK

NC = 2
NS = 16
NW = NC * NS
TPW = S // NW


def _router_body(hs_ref, gw_ref, pos1_ref, pos2_ref, w1_ref, w2_ref, be_ref):
    x = hs_ref[...]
    gw = gw_ref[...]
    logits = lax.dot_general(x, gw, (((1,), (1,)), ((), ())),
                             preferred_element_type=jnp.float32)

    eio = lax.broadcasted_iota(jnp.int32, (S, E), 1)
    m1 = jnp.max(logits, axis=1, keepdims=True)
    i1 = jnp.min(jnp.where(logits == m1, eio, E), axis=1, keepdims=True)
    rest = jnp.where(eio == i1, -jnp.inf, logits)
    m2 = jnp.max(rest, axis=1, keepdims=True)
    i2 = jnp.min(jnp.where(rest == m2, eio, E), axis=1, keepdims=True)

    w1 = 1.0 / (1.0 + jnp.exp(m2 - m1))
    w1_ref[...] = w1
    w2_ref[...] = 1.0 - w1

    mask1 = (eio == i1).astype(jnp.float32)
    mask2 = (eio == i2).astype(jnp.float32)

    rio = lax.broadcasted_iota(jnp.int32, (S, S), 0)
    cio = lax.broadcasted_iota(jnp.int32, (S, S), 1)
    tri = (rio >= cio).astype(jnp.float32)
    cum1 = lax.dot_general(tri, mask1, (((1,), (0,)), ((), ())),
                           preferred_element_type=jnp.float32)
    cum2 = lax.dot_general(tri, mask2, (((1,), (0,)), ((), ())),
                           preferred_element_type=jnp.float32)

    c1 = jnp.sum(mask1, axis=0, keepdims=True)
    c2 = jnp.sum(mask2, axis=0, keepdims=True)
    loc1 = cum1 - 1.0
    loc2 = cum2 - 1.0 + c1
    total = (c1 + c2).astype(jnp.int32)
    padded = ((total + (BLK - 1)) // BLK) * BLK
    eri = lax.broadcasted_iota(jnp.int32, (E, E), 0)
    eci = lax.broadcasted_iota(jnp.int32, (E, E), 1)
    etri = (eri < eci).astype(jnp.float32)
    off = lax.dot_general(padded.astype(jnp.float32), etri,
                          (((1,), (0,)), ((), ())),
                          preferred_element_type=jnp.float32)

    pos1_ref[...] = jnp.sum(mask1 * (off + loc1), axis=1,
                            keepdims=True).astype(jnp.int32)
    pos2_ref[...] = jnp.sum(mask2 * (off + loc2), axis=1,
                            keepdims=True).astype(jnp.int32)

    ends = (off.astype(jnp.int32) + padded) // BLK
    bio = lax.broadcasted_iota(jnp.int32, (NBLK, E), 0)
    be = jnp.sum((jnp.broadcast_to(ends, (NBLK, E)) <= bio).astype(jnp.int32),
                 axis=1, keepdims=True)
    be_ref[...] = jnp.minimum(be, E - 1)


_router = pl.pallas_call(
    _router_body,
    out_shape=[
        jax.ShapeDtypeStruct((S, 1), jnp.int32),
        jax.ShapeDtypeStruct((S, 1), jnp.int32),
        jax.ShapeDtypeStruct((S, 1), jnp.float32),
        jax.ShapeDtypeStruct((S, 1), jnp.float32),
        jax.ShapeDtypeStruct((NBLK, 1), jnp.int32),
    ],
)


@functools.lru_cache(maxsize=None)
def _sc_kernels():
    mesh = plsc.VectorSubcoreMesh(core_axis_name="c", subcore_axis_name="s",
                                  num_cores=NC, num_subcores=NS)

    @functools.partial(
        pl.kernel,
        mesh=mesh,
        out_type=jax.ShapeDtypeStruct((CAP, H), jnp.float32),
        scratch_types=[
            pltpu.VMEM((TPW,), jnp.int32),
            pltpu.VMEM((TPW,), jnp.int32),
            pltpu.VMEM((TPW, H), jnp.float32),
            pltpu.SemaphoreType.DMA,
        ],
    )
    def dispatch(hs_hbm, pos1_hbm, pos2_hbm, disp_hbm, idx1_v, idx2_v, rows_v,
                 sem):
        wid = lax.axis_index("s") * NC + lax.axis_index("c")
        base = wid * TPW
        pltpu.sync_copy(hs_hbm.at[pl.ds(base, TPW)], rows_v)
        pltpu.sync_copy(pos1_hbm.at[pl.ds(base, TPW)], idx1_v)
        pltpu.sync_copy(pos2_hbm.at[pl.ds(base, TPW)], idx2_v)
        pltpu.async_copy(rows_v, disp_hbm.at[idx1_v], sem).wait()
        pltpu.async_copy(rows_v, disp_hbm.at[idx2_v], sem).wait()

    @functools.partial(
        pl.kernel,
        mesh=mesh,
        out_type=jax.ShapeDtypeStruct((2 * S, H), jnp.float32),
        scratch_types=[
            pltpu.VMEM((TPW,), jnp.int32),
            pltpu.VMEM((TPW, H), jnp.float32),
            pltpu.SemaphoreType.DMA,
        ],
    )
    def gather_back(y_hbm, pos1_hbm, pos2_hbm, yg_hbm, idx_v, rows_v, sem):
        wid = lax.axis_index("s") * NC + lax.axis_index("c")
        base = wid * TPW
        pltpu.sync_copy(pos1_hbm.at[pl.ds(base, TPW)], idx_v)
        pltpu.async_copy(y_hbm.at[idx_v], rows_v, sem).wait()
        pltpu.sync_copy(rows_v, yg_hbm.at[pl.ds(base, TPW)])
        pltpu.sync_copy(pos2_hbm.at[pl.ds(base, TPW)], idx_v)
        pltpu.async_copy(y_hbm.at[idx_v], rows_v, sem).wait()
        pltpu.sync_copy(rows_v, yg_hbm.at[pl.ds(S + base, TPW)])

    return dispatch, gather_back


def _ffn_body(be_ref, x_ref, w13_ref, w2_ref, y_ref):
    x = x_ref[...]
    w13 = w13_ref[0]
    gu = lax.dot_general(x, w13, (((1,), (1,)), ((), ())),
                         preferred_element_type=jnp.float32)
    g = gu[:, :I]
    u = gu[:, I:]
    act = (g * jax.nn.sigmoid(g)) * u
    w2e = w2_ref[0]
    y_ref[...] = lax.dot_general(act, w2e, (((1,), (1,)), ((), ())),
                                 preferred_element_type=jnp.float32)


_ffn = pl.pallas_call(
    _ffn_body,
    grid_spec=pltpu.PrefetchScalarGridSpec(
        num_scalar_prefetch=1,
        grid=(NBLK,),
        in_specs=[
            pl.BlockSpec((BLK, H), lambda i, be: (i, 0)),
            pl.BlockSpec((1, 2 * I, H), lambda i, be: (be[i], 0, 0)),
            pl.BlockSpec((1, H, I), lambda i, be: (be[i], 0, 0)),
        ],
        out_specs=pl.BlockSpec((BLK, H), lambda i, be: (i, 0)),
    ),
    out_shape=jax.ShapeDtypeStruct((CAP, H), jnp.float32),
    compiler_params=pltpu.CompilerParams(
        dimension_semantics=("arbitrary",),
        vmem_limit_bytes=128 * 1024 * 1024,
    ),
)


def _combine_body(y1_ref, y2_ref, w1_ref, w2_ref, o_ref):
    o_ref[...] = w1_ref[...] * y1_ref[...] + w2_ref[...] * y2_ref[...]


_NCOMB = 8

_combine = pl.pallas_call(
    _combine_body,
    grid=(_NCOMB,),
    in_specs=[
        pl.BlockSpec((S // _NCOMB, H), lambda i: (i, 0)),
        pl.BlockSpec((S // _NCOMB, H), lambda i: (i + _NCOMB, 0)),
        pl.BlockSpec((S // _NCOMB, 1), lambda i: (i, 0)),
        pl.BlockSpec((S // _NCOMB, 1), lambda i: (i, 0)),
    ],
    out_specs=pl.BlockSpec((S // _NCOMB, H), lambda i: (i, 0)),
    out_shape=jax.ShapeDtypeStruct((S, H), jnp.float32),
)


def kernel(hidden_states, gate_weight, w13_weight, w2_weight):
    dispatch, gather_back = _sc_kernels()
    pos1, pos2, w1, w2, be = _router(hidden_states, gate_weight)
    pos1 = pos1.reshape(S)
    pos2 = pos2.reshape(S)
    disp = dispatch(hidden_states, pos1, pos2)
    y = _ffn(be.reshape(NBLK), disp, w13_weight, w2_weight)
    yg = gather_back(y, pos1, pos2)
    return _combine(yg, yg, w1, w2)
